# Initial kernel scaffold; baseline (speedup 1.0000x reference)
import jax
import jax.numpy as jnp
from jax.experimental import pallas as pl


def kernel(x, edge_attr, We, be, W1, b1, g1, bt1, W2, b2, g2, bt2, W3, b3, g3, bt3, W4, b4, g4, bt4, W5, b5, g5, bt5, edge_index):
    raise NotImplementedError("write your pallas kernel here")


# speedup vs baseline: 1.5881x; 1.5881x over previous
import functools

import jax
import jax.numpy as jnp
from jax import lax
from jax.experimental import pallas as pl
from jax.experimental.pallas import tpu as pltpu
from jax.experimental.pallas import tpu_sc as plsc

N = 10000
E = 320000
D = 128

NC = 2
NS = 16
CH = 128

NW = NC * NS
CPW = -(-E // (NW * CH))
EPW = CPW * CH
EP = EPW * NW
ZR = -(-(N // NS + 1) // 8) * 8
N2 = ZR * NS


def _sc_aggr_body(h_hbm, srcp, dstp, ap, w_hbm, be_hbm, zeros_hbm, out_hbm,
                  idx_v, dst_v, a_v, rows_v, wv, bev, acc, sem):
    c = lax.axis_index("c")
    s = lax.axis_index("s")

    pltpu.sync_copy(zeros_hbm, acc.at[pl.ds(s * ZR, ZR)])
    pltpu.sync_copy(w_hbm, wv)
    pltpu.sync_copy(be_hbm, bev)
    plsc.subcore_barrier()

    ebase = (c * NS + s) * EPW

    def chunk_body(ci, carry):
        off = ebase + ci * CH
        pltpu.sync_copy(srcp.at[pl.ds(off, CH)], idx_v)
        pltpu.sync_copy(dstp.at[pl.ds(off, CH)], dst_v)
        pltpu.sync_copy(ap.at[pl.ds(off, CH)], a_v)
        pltpu.async_copy(h_hbm.at[idx_v], rows_v, sem).wait()

        def group_body(gi, carry2):
            a16 = a_v[pl.ds(gi * 16, 16)]
            for j in range(16):
                a = a16[j]
                row = gi * 16 + j
                for f in range(D // 16):
                    sl = pl.ds(f * 16, 16)
                    r = rows_v[row, sl]
                    rows_v[row, sl] = jnp.maximum(r + a * wv[sl] + bev[sl], 0.0)
            return carry2

        lax.fori_loop(0, CH // 16, group_body, 0)
        pltpu.sync_copy(rows_v, acc.at[dst_v], add=True)
        return carry

    lax.fori_loop(0, CPW, chunk_body, 0)
    plsc.subcore_barrier()

    pltpu.sync_copy(acc.at[pl.ds(s * ZR, ZR)], out_hbm.at[c, pl.ds(s * ZR, ZR)])


@jax.jit
def _sc_aggr_call(h, srcp, dstp, ap, w, be, zeros):
    mesh = plsc.VectorSubcoreMesh(core_axis_name="c", subcore_axis_name="s")
    return pl.kernel(
        _sc_aggr_body,
        out_type=jax.ShapeDtypeStruct((NC, N2, D), jnp.float32),
        mesh=mesh,
        scratch_types=[
            pltpu.VMEM((CH,), jnp.int32),
            pltpu.VMEM((CH,), jnp.int32),
            pltpu.VMEM((CH,), jnp.float32),
            pltpu.VMEM((CH, D), jnp.float32),
            pltpu.VMEM((D,), jnp.float32),
            pltpu.VMEM((D,), jnp.float32),
            pltpu.VMEM_SHARED((N2, D), jnp.float32),
            pltpu.SemaphoreType.DMA,
        ],
    )(h, srcp, dstp, ap, w, be, zeros)


def _bn(y, g, bt):
    mu = jnp.mean(y, axis=0, keepdims=True)
    var = jnp.mean((y - mu) ** 2, axis=0, keepdims=True)
    return (y - mu) * lax.rsqrt(var + 1e-5) * g + bt


def _tc1_body(x_ref, agg_ref, W1_ref, b1_ref, g1_ref, bt1_ref, o_ref):
    agg = agg_ref[...]
    t = x_ref[...] + agg[0, :N, :] + agg[1, :N, :]
    y = jnp.maximum(
        jnp.dot(t, W1_ref[...], preferred_element_type=jnp.float32) + b1_ref[...],
        0.0)
    o_ref[...] = _bn(y, g1_ref[...], bt1_ref[...])


@jax.jit
def _tc1_call(x, agg, W1, b1, g1, bt1):
    return pl.pallas_call(
        _tc1_body,
        out_shape=jax.ShapeDtypeStruct((N, D), jnp.float32),
    )(x, agg, W1, b1, g1, bt1)


def _tc2_body(x1_ref, agg_ref, W2_ref, b2_ref, g2_ref, bt2_ref,
              W3a_ref, W3b_ref, b3_ref, g3_ref, bt3_ref,
              W4_ref, b4_ref, g4_ref, bt4_ref,
              W5_ref, b5_ref, g5_ref, bt5_ref, o_ref):
    x1 = x1_ref[...]
    agg = agg_ref[...]
    t = x1 + agg[0, :N, :] + agg[1, :N, :]
    y = jnp.maximum(
        jnp.dot(t, W2_ref[...], preferred_element_type=jnp.float32) + b2_ref[...],
        0.0)
    x2 = _bn(y, g2_ref[...], bt2_ref[...])
    h = (jnp.dot(x1, W3a_ref[...], preferred_element_type=jnp.float32)
         + jnp.dot(x2, W3b_ref[...], preferred_element_type=jnp.float32)
         + b3_ref[...])
    h = _bn(jnp.maximum(h, 0.0), g3_ref[...], bt3_ref[...])
    h = jnp.maximum(
        jnp.dot(h, W4_ref[...], preferred_element_type=jnp.float32) + b4_ref[...],
        0.0)
    h = _bn(h, g4_ref[...], bt4_ref[...])
    h = jnp.maximum(
        jnp.dot(h, W5_ref[...], preferred_element_type=jnp.float32) + b5_ref[...],
        0.0)
    o_ref[...] = _bn(h, g5_ref[...], bt5_ref[...])


@jax.jit
def _tc2_call(x1, agg, W2, b2, g2, bt2, W3a, W3b, b3, g3, bt3,
              W4, b4, g4, bt4, W5, b5, g5, bt5):
    return pl.pallas_call(
        _tc2_body,
        out_shape=jax.ShapeDtypeStruct((N, 16), jnp.float32),
    )(x1, agg, W2, b2, g2, bt2, W3a, W3b, b3, g3, bt3,
      W4, b4, g4, bt4, W5, b5, g5, bt5)


def kernel(x, edge_attr, We, be, W1, b1, g1, bt1, W2, b2, g2, bt2,
           W3, b3, g3, bt3, W4, b4, g4, bt4, W5, b5, g5, bt5, edge_index):
    src = edge_index[0]
    dst = edge_index[1]
    a = edge_attr[:, 0]

    pad = EP - E
    srcp = jnp.concatenate([src, jnp.zeros((pad,), jnp.int32)])
    dstp = jnp.concatenate([dst, jnp.full((pad,), N, jnp.int32)])
    ap = jnp.concatenate([a, jnp.zeros((pad,), jnp.float32)])

    w = We[0]
    zeros = jnp.zeros((ZR, D), jnp.float32)

    r1 = lambda v: v.reshape(1, -1)

    agg1 = _sc_aggr_call(x, srcp, dstp, ap, w, be, zeros)
    x1 = _tc1_call(x, agg1, W1, r1(b1), r1(g1), r1(bt1))
    agg2 = _sc_aggr_call(x1, srcp, dstp, ap, w, be, zeros)
    out = _tc2_call(x1, agg2, W2, r1(b2), r1(g2), r1(bt2),
                    W3[:D], W3[D:], r1(b3), r1(g3), r1(bt3),
                    W4, r1(b4), r1(g4), r1(bt4),
                    W5, r1(b5), r1(g5), r1(bt5))
    return out
